# baseline (speedup 1.0000x reference)
import functools
import jax
import jax.numpy as jnp
from jax import lax
from jax.experimental import pallas as pl
from jax.experimental.pallas import tpu as pltpu
from jax.experimental.pallas import tpu_sc as plsc

N = 10000
E = 320000
H = 128
NH = 8
HD = 16
NRBF = 32
CUT_UPPER = 5.0

BN = 1000
NB_N = N // BN
BE = 512
NB_E = E // BE


def _silu(x):
    return x * jax.nn.sigmoid(x)


def _node_pre_body(x_ref, vec_ref, lnw_ref, lnb_ref, wq_ref, bq_ref,
                   wk_ref, bk_ref, wv2_ref, bv2_ref, wvec_ref,
                   q_ref, kvv_ref, vec3_ref, vdot_ref):
    x = x_ref[...]
    mu = jnp.mean(x, axis=-1, keepdims=True)
    var = jnp.mean((x - mu) ** 2, axis=-1, keepdims=True)
    xn = (x - mu) * lax.rsqrt(var + 1e-5) * lnw_ref[...] + lnb_ref[...]
    q = jnp.dot(xn, wq_ref[...].T, preferred_element_type=jnp.float32) + bq_ref[...]
    k = jnp.dot(xn, wk_ref[...].T, preferred_element_type=jnp.float32) + bk_ref[...]
    v = jnp.dot(xn, wv2_ref[...].T, preferred_element_type=jnp.float32) + bv2_ref[...]
    vec = vec_ref[...]
    vecf = vec.reshape(BN * 3, H)
    vp = jnp.dot(vecf, wvec_ref[...].T, preferred_element_type=jnp.float32)
    vp = vp.reshape(BN, 3, 3 * H)
    vec1 = vp[:, :, :H]
    vec2 = vp[:, :, H:2 * H]
    vec3 = vp[:, :, 2 * H:]
    vdot_ref[...] = jnp.sum(vec1 * vec2, axis=1)
    vec3_ref[...] = vec3.reshape(BN, 3 * H)
    q_ref[...] = q
    kvlo = jnp.concatenate([k, v], axis=-1)
    kvhi = jnp.concatenate([vec.reshape(BN, 3 * H),
                            jnp.zeros((BN, H), jnp.float32)], axis=-1)
    lo_u = lax.bitcast_convert_type(
        kvlo.astype(jnp.bfloat16).astype(jnp.float32), jnp.uint32)
    hi_u = lax.bitcast_convert_type(
        kvhi.astype(jnp.bfloat16).astype(jnp.float32), jnp.uint32)
    kvv_ref[...] = lax.bitcast_convert_type(hi_u | (lo_u >> 16), jnp.int32)


def _node_pre(x, vec, ln_w, ln_b, Wq, bq, Wk, bk, Wv2, bv2, Wvec):
    wspec = lambda shp: pl.BlockSpec(shp, lambda i: (0,) * len(shp))
    return pl.pallas_call(
        _node_pre_body,
        grid=(NB_N,),
        in_specs=[
            pl.BlockSpec((BN, H), lambda i: (i, 0)),
            pl.BlockSpec((BN, 3, H), lambda i: (i, 0, 0)),
            wspec((H,)), wspec((H,)),
            wspec((H, H)), wspec((H,)),
            wspec((H, H)), wspec((H,)),
            wspec((3 * H, H)), wspec((3 * H,)),
            wspec((3 * H, H)),
        ],
        out_specs=[
            pl.BlockSpec((BN, H), lambda i: (i, 0)),
            pl.BlockSpec((BN, 4 * H), lambda i: (i, 0)),
            pl.BlockSpec((BN, 3 * H), lambda i: (i, 0)),
            pl.BlockSpec((BN, H), lambda i: (i, 0)),
        ],
        out_shape=[
            jax.ShapeDtypeStruct((N, H), jnp.float32),
            jax.ShapeDtypeStruct((N, 4 * H), jnp.int32),
            jax.ShapeDtypeStruct((N, 3 * H), jnp.float32),
            jax.ShapeDtypeStruct((N, H), jnp.float32),
        ],
    )(x, vec, ln_w, ln_b, Wq, bq, Wk, bk, Wv2, bv2, Wvec)


def _edge_msg_body(qg_ref, kvvg_ref, f_ref, rd_ref, wdk_ref, bdk_ref,
                   wdv2_ref, bdv2_ref, m0_ref, m1_ref, m2_ref, m3_ref):
    f = f_ref[...]
    dk = _silu(lax.dot_general(f, wdk_ref[...], (((1,), (1,)), ((), ())),
                               preferred_element_type=jnp.float32) + bdk_ref[...])
    dv = _silu(lax.dot_general(f, wdv2_ref[...], (((1,), (1,)), ((), ())),
                               preferred_element_type=jnp.float32) + bdv2_ref[...])
    qg = qg_ref[...]
    pk = lax.bitcast_convert_type(kvvg_ref[...], jnp.uint32)
    lo = lax.bitcast_convert_type(pk << 16, jnp.float32)
    hi = lax.bitcast_convert_type(pk & jnp.uint32(0xFFFF0000),
                                  jnp.float32)
    kg = lo[:, :H]
    vg = lo[:, H:4 * H]
    vecg = hi
    s = qg * kg * dk
    ri = lax.broadcasted_iota(jnp.int32, (H, H), 0) // HD
    ci = lax.broadcasted_iota(jnp.int32, (H, H), 1) // HD
    M = (ri == ci).astype(jnp.float32)
    attn = jnp.dot(s, M, preferred_element_type=jnp.float32)
    rd = rd_ref[...]
    r = rd[:, 0:1]
    cut = 0.5 * (jnp.cos(r * (jnp.pi / CUT_UPPER)) + 1.0)
    cut = cut * (r < CUT_UPPER).astype(jnp.float32)
    attn = _silu(attn) * cut
    m0_ref[...] = vg[:, :H] * dv[:, :H] * attn
    vm1 = vg[:, H:2 * H] * dv[:, H:2 * H]
    vm2 = vg[:, 2 * H:] * dv[:, 2 * H:]
    m1_ref[...] = vecg[:, :H] * vm1 + vm2 * rd[:, 1:2]
    m2_ref[...] = vecg[:, H:2 * H] * vm1 + vm2 * rd[:, 2:3]
    m3_ref[...] = vecg[:, 2 * H:3 * H] * vm1 + vm2 * rd[:, 3:4]


def _edge_messages(qg, kvvg, f_ij, rd, Wdk, bdk, Wdv2, bdv2, eh, off_b):
    wspec = lambda shp: pl.BlockSpec(shp, lambda i: (0,) * len(shp))
    espec = lambda w: pl.BlockSpec((BE, w), lambda i: (i + off_b, 0))
    return pl.pallas_call(
        _edge_msg_body,
        grid=(eh // BE,),
        in_specs=[
            espec(H), espec(4 * H), espec(NRBF), espec(4),
            wspec((H, NRBF)), wspec((H,)),
            wspec((3 * H, NRBF)), wspec((3 * H,)),
        ],
        out_specs=[pl.BlockSpec((BE, H), lambda i: (i, 0))] * 4,
        out_shape=[jax.ShapeDtypeStruct((eh, H), jnp.float32)] * 4,
    )(qg, kvvg, f_ij, rd, Wdk, bdk, Wdv2, bdv2)


NW_G = 32
CH_G = 40


def _sc_gather(q, kvv, src, dst, eh):
    epw = eh // NW_G
    npair = epw // (2 * CH_G)

    def body(q_h, kvv_h, src_h, dst_h, qg_h, kvvg_h,
             dstv, srcv, qb0, qb1, kb0, kb1,
             gsq, gsk, wsq0, wsq1, wsk0, wsk1):
        c = lax.axis_index("c")
        s = lax.axis_index("s")
        base = (s * 2 + c) * epw
        pltpu.sync_copy(dst_h.at[pl.ds(base, epw)], dstv)
        pltpu.sync_copy(src_h.at[pl.ds(base, epw)], srcv)

        def pair(g, qb, kb, wsq, wsk, b):
            off = (2 * g + b) * CH_G
            eb = base + off

            @pl.when(g > 0)
            def _():
                pltpu.make_async_copy(qb, qg_h.at[pl.ds(eb, CH_G), :], wsq).wait()
                pltpu.make_async_copy(kb, kvvg_h.at[pl.ds(eb, CH_G), :], wsk).wait()

            hq = pltpu.async_copy(q_h.at[dstv.at[pl.ds(off, CH_G)]], qb, gsq)
            hk = pltpu.async_copy(kvv_h.at[srcv.at[pl.ds(off, CH_G)]], kb, gsk)
            hq.wait()
            hk.wait()
            pltpu.async_copy(qb, qg_h.at[pl.ds(eb, CH_G), :], wsq)
            pltpu.async_copy(kb, kvvg_h.at[pl.ds(eb, CH_G), :], wsk)

        def step(g, carry):
            pair(g, qb0, kb0, wsq0, wsk0, 0)
            pair(g, qb1, kb1, wsq1, wsk1, 1)
            return carry
        lax.fori_loop(0, npair, step, 0)

        pltpu.make_async_copy(qb0, qg_h.at[pl.ds(base, CH_G), :], wsq0).wait()
        pltpu.make_async_copy(kb0, kvvg_h.at[pl.ds(base, CH_G), :], wsk0).wait()
        pltpu.make_async_copy(qb1, qg_h.at[pl.ds(base, CH_G), :], wsq1).wait()
        pltpu.make_async_copy(kb1, kvvg_h.at[pl.ds(base, CH_G), :], wsk1).wait()

    mesh = plsc.VectorSubcoreMesh(core_axis_name="c", subcore_axis_name="s")
    f = pl.kernel(
        body,
        out_type=[
            jax.ShapeDtypeStruct((eh, H), jnp.float32),
            jax.ShapeDtypeStruct((eh, 4 * H), jnp.int32),
        ],
        mesh=mesh,
        scratch_types=[
            pltpu.VMEM((epw,), jnp.int32),
            pltpu.VMEM((epw,), jnp.int32),
            pltpu.VMEM((CH_G, H), jnp.float32),
            pltpu.VMEM((CH_G, H), jnp.float32),
            pltpu.VMEM((CH_G, 4 * H), jnp.int32),
            pltpu.VMEM((CH_G, 4 * H), jnp.int32),
            pltpu.SemaphoreType.DMA,
            pltpu.SemaphoreType.DMA,
            pltpu.SemaphoreType.DMA,
            pltpu.SemaphoreType.DMA,
            pltpu.SemaphoreType.DMA,
            pltpu.SemaphoreType.DMA,
        ],
    )
    return f(q, kvv, src, dst)


NS_SC = 16
CH_S = 80
N_PAD = 10240
ZR = N_PAD // NS_SC
TAIL_R = N - (NS_SC - 1) * ZR


def _sc_scatter(m0, m1, m2, m3, i0, i1, i2, i3, dst, eh):
    ept = eh // NS_SC
    nch = ept // CH_S

    def body(m0, m1, m2, m3, i0r, i1r, i2r, i3r, dstr,
             a0, a1, a2, a3, acc, mb0, mb1, ib0, ib1, ls0, ls1):
        c = lax.axis_index("c")
        s = lax.axis_index("s")

        def do_slice(m_ref, init_ref, out_ref):
            @pl.when(s < NS_SC - 1)
            def _():
                pltpu.sync_copy(init_ref.at[pl.ds(s * ZR, ZR), :],
                                acc.at[pl.ds(s * ZR, ZR), :])

            @pl.when(s == NS_SC - 1)
            def _():
                pltpu.sync_copy(init_ref.at[pl.ds(s * ZR, TAIL_R), :],
                                acc.at[pl.ds(s * ZR, TAIL_R), :])
            plsc.subcore_barrier()

            tbase = s * ept

            def load(i, mb, ib, ls):
                eb = tbase + i * CH_S
                pltpu.async_copy(dstr.at[pl.ds(eb, CH_S)], ib, ls)
                pltpu.async_copy(m_ref.at[pl.ds(eb, CH_S), :], mb, ls)

            def drain(i, mb, ib, ls):
                eb = tbase + i * CH_S
                pltpu.make_async_copy(dstr.at[pl.ds(eb, CH_S)], ib, ls).wait()
                pltpu.make_async_copy(m_ref.at[pl.ds(eb, CH_S), :], mb, ls).wait()

            load(0, mb0, ib0, ls0)
            load(1, mb1, ib1, ls1)

            def step(i, mb, ib, ls, b):
                drain(2 * i + b, mb, ib, ls)
                pltpu.sync_copy(mb, acc.at[ib], add=True)

                @pl.when(i < nch // 2 - 1)
                def _():
                    load(2 * i + b + 2, mb, ib, ls)

            def _chunk(i, carry):
                step(i, mb0, ib0, ls0, 0)
                step(i, mb1, ib1, ls1, 1)
                return carry
            lax.fori_loop(0, nch // 2, _chunk, 0)
            plsc.subcore_barrier()

            @pl.when(s < NS_SC - 1)
            def _full():
                pltpu.sync_copy(acc.at[pl.ds(s * ZR, ZR), :],
                                out_ref.at[pl.ds(s * ZR, ZR), :])

            @pl.when(s == NS_SC - 1)
            def _tail():
                pltpu.sync_copy(acc.at[pl.ds(s * ZR, TAIL_R), :],
                                out_ref.at[pl.ds(s * ZR, TAIL_R), :])

        @pl.when(c == 0)
        def _():
            do_slice(m0, i0r, a0)
            do_slice(m2, i2r, a2)

        @pl.when(c == 1)
        def _():
            do_slice(m1, i1r, a1)
            do_slice(m3, i3r, a3)

    mesh = plsc.VectorSubcoreMesh(core_axis_name="c", subcore_axis_name="s")
    f = pl.kernel(
        body,
        out_type=[jax.ShapeDtypeStruct((N, H), jnp.float32)] * 4,
        mesh=mesh,
        scratch_types=[
            pltpu.VMEM_SHARED((N_PAD, H), jnp.float32),
            pltpu.VMEM((CH_S, H), jnp.float32),
            pltpu.VMEM((CH_S, H), jnp.float32),
            pltpu.VMEM((CH_S,), jnp.int32),
            pltpu.VMEM((CH_S,), jnp.int32),
            pltpu.SemaphoreType.DMA,
            pltpu.SemaphoreType.DMA,
        ],
    )
    return f(m0, m1, m2, m3, i0, i1, i2, i3, dst)


def _node_post_body(a0_ref, a1_ref, a2_ref, a3_ref, vdot_ref, vec3_ref,
                    wo_ref, bo_ref, dx_ref, dvec_ref):
    o = jnp.dot(a0_ref[...], wo_ref[...].T, preferred_element_type=jnp.float32) + bo_ref[...]
    o1 = o[:, :H]
    o2 = o[:, H:2 * H]
    o3 = o[:, 2 * H:]
    dx_ref[...] = vdot_ref[...] * o2 + o3
    vec3 = vec3_ref[...]
    d1 = vec3[:, :H] * o1 + a1_ref[...]
    d2 = vec3[:, H:2 * H] * o1 + a2_ref[...]
    d3 = vec3[:, 2 * H:] * o1 + a3_ref[...]
    dvec_ref[...] = jnp.concatenate([d1, d2, d3], axis=-1)


def _node_post(a0, a1, a2, a3, vdot, vec3, Wo, bo):
    wspec = lambda shp: pl.BlockSpec(shp, lambda i: (0,) * len(shp))
    nspec = lambda w: pl.BlockSpec((BN, w), lambda i: (i, 0))
    return pl.pallas_call(
        _node_post_body,
        grid=(NB_N,),
        in_specs=[
            nspec(H), nspec(H), nspec(H), nspec(H), nspec(H), nspec(3 * H),
            wspec((3 * H, H)), wspec((3 * H,)),
        ],
        out_specs=[nspec(H), nspec(3 * H)],
        out_shape=[
            jax.ShapeDtypeStruct((N, H), jnp.float32),
            jax.ShapeDtypeStruct((N, 3 * H), jnp.float32),
        ],
    )(a0, a1, a2, a3, vdot, vec3, Wo, bo)


def kernel(x, vec, edge_index, r_ij, f_ij, d_ij, ln_w, ln_b, Wq, bq, Wk, bk,
           Wv, bv, Wo, bo, Wvec, Wdk, bdk, Wdv, bdv):
    src = edge_index[0].astype(jnp.int32)
    dst = edge_index[1].astype(jnp.int32)

    h = jnp.arange(NH, dtype=jnp.int32)[:, None] * (3 * HD)
    d = jnp.arange(HD, dtype=jnp.int32)[None, :]
    perm = jnp.concatenate([(h + d).reshape(-1), (h + HD + d).reshape(-1),
                            (h + 2 * HD + d).reshape(-1)])
    Wv2 = Wv[perm]
    bv2 = bv[perm]
    Wdv2 = Wdv[perm]
    bdv2 = bdv[perm]

    q, kvv, vec3, vdot = _node_pre(x, vec, ln_w, ln_b, Wq, bq, Wk, bk,
                                   Wv2, bv2, Wvec)

    rd = jnp.concatenate([r_ij[:, None], d_ij], axis=1)

    EA = 158720
    EB = E - EA
    z = jnp.zeros((N, H), jnp.float32)

    qg, kvvg = _sc_gather(q, kvv, src, dst, E)
    ma = _edge_messages(qg, kvvg, f_ij, rd, Wdk, bdk,
                        Wdv2, bdv2, EA, 0)
    mb = _edge_messages(qg, kvvg, f_ij, rd, Wdk, bdk,
                        Wdv2, bdv2, EB, EA // BE)
    p0, p1, p2, p3 = _sc_scatter(ma[0], ma[1], ma[2], ma[3],
                                 z, z, z, z, dst[:EA], EA)
    a0, a1, a2, a3 = _sc_scatter(mb[0], mb[1], mb[2], mb[3],
                                 p0, p1, p2, p3, dst[EA:], EB)

    dx, dvec_flat = _node_post(a0, a1, a2, a3, vdot, vec3, Wo, bo)
    return dx, dvec_flat.reshape(N, 3, H)


# speedup vs baseline: 1.0911x; 1.0911x over previous
import functools
import jax
import jax.numpy as jnp
from jax import lax
from jax.experimental import pallas as pl
from jax.experimental.pallas import tpu as pltpu
from jax.experimental.pallas import tpu_sc as plsc

N = 10000
E = 320000
H = 128
NH = 8
HD = 16
NRBF = 32
CUT_UPPER = 5.0

BN = 1000
NB_N = N // BN
BE = 512
NB_E = E // BE


def _silu(x):
    return x * jax.nn.sigmoid(x)


def _node_pre_body(x_ref, vec_ref, lnw_ref, lnb_ref, wq_ref, bq_ref,
                   wk_ref, bk_ref, wv2_ref, bv2_ref, wvec_ref,
                   q_ref, kvv_ref, vec3_ref, vdot_ref):
    x = x_ref[...]
    mu = jnp.mean(x, axis=-1, keepdims=True)
    var = jnp.mean((x - mu) ** 2, axis=-1, keepdims=True)
    xn = (x - mu) * lax.rsqrt(var + 1e-5) * lnw_ref[...] + lnb_ref[...]
    q = jnp.dot(xn, wq_ref[...].T, preferred_element_type=jnp.float32) + bq_ref[...]
    k = jnp.dot(xn, wk_ref[...].T, preferred_element_type=jnp.float32) + bk_ref[...]
    v = jnp.dot(xn, wv2_ref[...].T, preferred_element_type=jnp.float32) + bv2_ref[...]
    vec = vec_ref[...]
    vecf = vec.reshape(BN * 3, H)
    vp = jnp.dot(vecf, wvec_ref[...].T, preferred_element_type=jnp.float32)
    vp = vp.reshape(BN, 3, 3 * H)
    vec1 = vp[:, :, :H]
    vec2 = vp[:, :, H:2 * H]
    vec3 = vp[:, :, 2 * H:]
    vdot_ref[...] = jnp.sum(vec1 * vec2, axis=1)
    vec3_ref[...] = vec3.reshape(BN, 3 * H)
    q_ref[...] = q
    kvlo = jnp.concatenate([k, v], axis=-1)
    kvhi = jnp.concatenate([vec.reshape(BN, 3 * H),
                            jnp.zeros((BN, H), jnp.float32)], axis=-1)
    lo_u = lax.bitcast_convert_type(
        kvlo.astype(jnp.bfloat16).astype(jnp.float32), jnp.uint32)
    hi_u = lax.bitcast_convert_type(
        kvhi.astype(jnp.bfloat16).astype(jnp.float32), jnp.uint32)
    kvv_ref[...] = lax.bitcast_convert_type(hi_u | (lo_u >> 16), jnp.int32)


def _node_pre(x, vec, ln_w, ln_b, Wq, bq, Wk, bk, Wv2, bv2, Wvec):
    wspec = lambda shp: pl.BlockSpec(shp, lambda i: (0,) * len(shp))
    return pl.pallas_call(
        _node_pre_body,
        grid=(NB_N,),
        in_specs=[
            pl.BlockSpec((BN, H), lambda i: (i, 0)),
            pl.BlockSpec((BN, 3, H), lambda i: (i, 0, 0)),
            wspec((H,)), wspec((H,)),
            wspec((H, H)), wspec((H,)),
            wspec((H, H)), wspec((H,)),
            wspec((3 * H, H)), wspec((3 * H,)),
            wspec((3 * H, H)),
        ],
        out_specs=[
            pl.BlockSpec((BN, H), lambda i: (i, 0)),
            pl.BlockSpec((BN, 4 * H), lambda i: (i, 0)),
            pl.BlockSpec((BN, 3 * H), lambda i: (i, 0)),
            pl.BlockSpec((BN, H), lambda i: (i, 0)),
        ],
        out_shape=[
            jax.ShapeDtypeStruct((N, H), jnp.float32),
            jax.ShapeDtypeStruct((N, 4 * H), jnp.int32),
            jax.ShapeDtypeStruct((N, 3 * H), jnp.float32),
            jax.ShapeDtypeStruct((N, H), jnp.float32),
        ],
    )(x, vec, ln_w, ln_b, Wq, bq, Wk, bk, Wv2, bv2, Wvec)


def _edge_msg_body(qg_ref, kvvg_ref, f_ref, rd_ref, wdk_ref, bdk_ref,
                   wdv2_ref, bdv2_ref, m0_ref, m1_ref, m2_ref, m3_ref):
    f = f_ref[...]
    dk = _silu(lax.dot_general(f, wdk_ref[...], (((1,), (1,)), ((), ())),
                               preferred_element_type=jnp.float32) + bdk_ref[...])
    dv = _silu(lax.dot_general(f, wdv2_ref[...], (((1,), (1,)), ((), ())),
                               preferred_element_type=jnp.float32) + bdv2_ref[...])
    qg = qg_ref[...]
    pk = lax.bitcast_convert_type(kvvg_ref[...], jnp.uint32)
    lo = lax.bitcast_convert_type(pk << 16, jnp.float32)
    hi = lax.bitcast_convert_type(pk & jnp.uint32(0xFFFF0000),
                                  jnp.float32)
    kg = lo[:, :H]
    vg = lo[:, H:4 * H]
    vecg = hi
    s = qg * kg * dk
    ri = lax.broadcasted_iota(jnp.int32, (H, H), 0) // HD
    ci = lax.broadcasted_iota(jnp.int32, (H, H), 1) // HD
    M = (ri == ci).astype(jnp.float32)
    attn = jnp.dot(s, M, preferred_element_type=jnp.float32)
    rd = rd_ref[...]
    r = rd[:, 0:1]
    cut = 0.5 * (jnp.cos(r * (jnp.pi / CUT_UPPER)) + 1.0)
    cut = cut * (r < CUT_UPPER).astype(jnp.float32)
    attn = _silu(attn) * cut
    m0_ref[...] = vg[:, :H] * dv[:, :H] * attn
    vm1 = vg[:, H:2 * H] * dv[:, H:2 * H]
    vm2 = vg[:, 2 * H:] * dv[:, 2 * H:]
    m1_ref[...] = vecg[:, :H] * vm1 + vm2 * rd[:, 1:2]
    m2_ref[...] = vecg[:, H:2 * H] * vm1 + vm2 * rd[:, 2:3]
    m3_ref[...] = vecg[:, 2 * H:3 * H] * vm1 + vm2 * rd[:, 3:4]


def _edge_messages(qg, kvvg, f_ij, rd, Wdk, bdk, Wdv2, bdv2, eh, off_b):
    wspec = lambda shp: pl.BlockSpec(shp, lambda i: (0,) * len(shp))
    espec = lambda w: pl.BlockSpec((BE, w), lambda i: (i + off_b, 0))
    return pl.pallas_call(
        _edge_msg_body,
        grid=(eh // BE,),
        in_specs=[
            espec(H), espec(4 * H), espec(NRBF), espec(4),
            wspec((H, NRBF)), wspec((H,)),
            wspec((3 * H, NRBF)), wspec((3 * H,)),
        ],
        out_specs=[pl.BlockSpec((BE, H), lambda i: (i, 0))] * 4,
        out_shape=[jax.ShapeDtypeStruct((eh, H), jnp.float32)] * 4,
    )(qg, kvvg, f_ij, rd, Wdk, bdk, Wdv2, bdv2)


NW_G = 32
CH_G = 40


def _sc_gather(q, kvv, src, dst, eh):
    epw = eh // NW_G
    npair = epw // (2 * CH_G)

    def body(q_h, kvv_h, src_h, dst_h, qg_h, kvvg_h,
             dstv, srcv, qb0, qb1, kb0, kb1,
             gsq, gsk, wsq0, wsq1, wsk0, wsk1):
        c = lax.axis_index("c")
        s = lax.axis_index("s")
        base = (s * 2 + c) * epw
        pltpu.sync_copy(dst_h.at[pl.ds(base, epw)], dstv)
        pltpu.sync_copy(src_h.at[pl.ds(base, epw)], srcv)

        def pair(g, qb, kb, wsq, wsk, b):
            off = (2 * g + b) * CH_G
            eb = base + off

            @pl.when(g > 0)
            def _():
                pltpu.make_async_copy(qb, qg_h.at[pl.ds(eb, CH_G), :], wsq).wait()
                pltpu.make_async_copy(kb, kvvg_h.at[pl.ds(eb, CH_G), :], wsk).wait()

            hq = pltpu.async_copy(q_h.at[dstv.at[pl.ds(off, CH_G)]], qb, gsq)
            hk = pltpu.async_copy(kvv_h.at[srcv.at[pl.ds(off, CH_G)]], kb, gsk)
            hq.wait()
            hk.wait()
            pltpu.async_copy(qb, qg_h.at[pl.ds(eb, CH_G), :], wsq)
            pltpu.async_copy(kb, kvvg_h.at[pl.ds(eb, CH_G), :], wsk)

        def step(g, carry):
            pair(g, qb0, kb0, wsq0, wsk0, 0)
            pair(g, qb1, kb1, wsq1, wsk1, 1)
            return carry
        lax.fori_loop(0, npair, step, 0)

        pltpu.make_async_copy(qb0, qg_h.at[pl.ds(base, CH_G), :], wsq0).wait()
        pltpu.make_async_copy(kb0, kvvg_h.at[pl.ds(base, CH_G), :], wsk0).wait()
        pltpu.make_async_copy(qb1, qg_h.at[pl.ds(base, CH_G), :], wsq1).wait()
        pltpu.make_async_copy(kb1, kvvg_h.at[pl.ds(base, CH_G), :], wsk1).wait()

    mesh = plsc.VectorSubcoreMesh(core_axis_name="c", subcore_axis_name="s")
    f = pl.kernel(
        body,
        out_type=[
            jax.ShapeDtypeStruct((eh, H), jnp.float32),
            jax.ShapeDtypeStruct((eh, 4 * H), jnp.int32),
        ],
        mesh=mesh,
        scratch_types=[
            pltpu.VMEM((epw,), jnp.int32),
            pltpu.VMEM((epw,), jnp.int32),
            pltpu.VMEM((CH_G, H), jnp.float32),
            pltpu.VMEM((CH_G, H), jnp.float32),
            pltpu.VMEM((CH_G, 4 * H), jnp.int32),
            pltpu.VMEM((CH_G, 4 * H), jnp.int32),
            pltpu.SemaphoreType.DMA,
            pltpu.SemaphoreType.DMA,
            pltpu.SemaphoreType.DMA,
            pltpu.SemaphoreType.DMA,
            pltpu.SemaphoreType.DMA,
            pltpu.SemaphoreType.DMA,
        ],
    )
    return f(q, kvv, src, dst)


NS_SC = 16
CH_S = 80
N_PAD = 10240
ZR = N_PAD // NS_SC
TAIL_R = N - (NS_SC - 1) * ZR


def _sc_scatter(m0, m1, m2, m3, i0, i1, i2, i3, dst, eh):
    ept = eh // NS_SC
    nch = ept // CH_S

    def body(m0, m1, m2, m3, i0r, i1r, i2r, i3r, dstr,
             a0, a1, a2, a3, acc, mb0, mb1, ib0, ib1, ls0, ls1):
        c = lax.axis_index("c")
        s = lax.axis_index("s")

        def do_slice(m_ref, init_ref, out_ref):
            @pl.when(s < NS_SC - 1)
            def _():
                pltpu.sync_copy(init_ref.at[pl.ds(s * ZR, ZR), :],
                                acc.at[pl.ds(s * ZR, ZR), :])

            @pl.when(s == NS_SC - 1)
            def _():
                pltpu.sync_copy(init_ref.at[pl.ds(s * ZR, TAIL_R), :],
                                acc.at[pl.ds(s * ZR, TAIL_R), :])
            plsc.subcore_barrier()

            tbase = s * ept

            def load(i, mb, ib, ls):
                eb = tbase + i * CH_S
                pltpu.async_copy(dstr.at[pl.ds(eb, CH_S)], ib, ls)
                pltpu.async_copy(m_ref.at[pl.ds(eb, CH_S), :], mb, ls)

            def drain(i, mb, ib, ls):
                eb = tbase + i * CH_S
                pltpu.make_async_copy(dstr.at[pl.ds(eb, CH_S)], ib, ls).wait()
                pltpu.make_async_copy(m_ref.at[pl.ds(eb, CH_S), :], mb, ls).wait()

            load(0, mb0, ib0, ls0)
            load(1, mb1, ib1, ls1)

            def step(i, mb, ib, ls, b):
                drain(2 * i + b, mb, ib, ls)
                pltpu.sync_copy(mb, acc.at[ib], add=True)

                @pl.when(i < nch // 2 - 1)
                def _():
                    load(2 * i + b + 2, mb, ib, ls)

            def _chunk(i, carry):
                step(i, mb0, ib0, ls0, 0)
                step(i, mb1, ib1, ls1, 1)
                return carry
            lax.fori_loop(0, nch // 2, _chunk, 0)
            plsc.subcore_barrier()

            @pl.when(s < NS_SC - 1)
            def _full():
                pltpu.sync_copy(acc.at[pl.ds(s * ZR, ZR), :],
                                out_ref.at[pl.ds(s * ZR, ZR), :])

            @pl.when(s == NS_SC - 1)
            def _tail():
                pltpu.sync_copy(acc.at[pl.ds(s * ZR, TAIL_R), :],
                                out_ref.at[pl.ds(s * ZR, TAIL_R), :])

        @pl.when(c == 0)
        def _():
            do_slice(m0, i0r, a0)
            do_slice(m2, i2r, a2)

        @pl.when(c == 1)
        def _():
            do_slice(m1, i1r, a1)
            do_slice(m3, i3r, a3)

    mesh = plsc.VectorSubcoreMesh(core_axis_name="c", subcore_axis_name="s")
    f = pl.kernel(
        body,
        out_type=[jax.ShapeDtypeStruct((N, H), jnp.float32)] * 4,
        mesh=mesh,
        scratch_types=[
            pltpu.VMEM_SHARED((N_PAD, H), jnp.float32),
            pltpu.VMEM((CH_S, H), jnp.float32),
            pltpu.VMEM((CH_S, H), jnp.float32),
            pltpu.VMEM((CH_S,), jnp.int32),
            pltpu.VMEM((CH_S,), jnp.int32),
            pltpu.SemaphoreType.DMA,
            pltpu.SemaphoreType.DMA,
        ],
    )
    return f(m0, m1, m2, m3, i0, i1, i2, i3, dst)


def _node_post_body(a0_ref, a1_ref, a2_ref, a3_ref, vdot_ref, vec3_ref,
                    wo_ref, bo_ref, dx_ref, dvec_ref):
    o = jnp.dot(a0_ref[...], wo_ref[...].T, preferred_element_type=jnp.float32) + bo_ref[...]
    o1 = o[:, :H]
    o2 = o[:, H:2 * H]
    o3 = o[:, 2 * H:]
    dx_ref[...] = vdot_ref[...] * o2 + o3
    vec3 = vec3_ref[...]
    d1 = vec3[:, :H] * o1 + a1_ref[...]
    d2 = vec3[:, H:2 * H] * o1 + a2_ref[...]
    d3 = vec3[:, 2 * H:] * o1 + a3_ref[...]
    dvec_ref[...] = jnp.concatenate([d1, d2, d3], axis=-1)


def _node_post(a0, a1, a2, a3, vdot, vec3, Wo, bo):
    wspec = lambda shp: pl.BlockSpec(shp, lambda i: (0,) * len(shp))
    nspec = lambda w: pl.BlockSpec((BN, w), lambda i: (i, 0))
    return pl.pallas_call(
        _node_post_body,
        grid=(NB_N,),
        in_specs=[
            nspec(H), nspec(H), nspec(H), nspec(H), nspec(H), nspec(3 * H),
            wspec((3 * H, H)), wspec((3 * H,)),
        ],
        out_specs=[nspec(H), nspec(3 * H)],
        out_shape=[
            jax.ShapeDtypeStruct((N, H), jnp.float32),
            jax.ShapeDtypeStruct((N, 3 * H), jnp.float32),
        ],
    )(a0, a1, a2, a3, vdot, vec3, Wo, bo)


def kernel(x, vec, edge_index, r_ij, f_ij, d_ij, ln_w, ln_b, Wq, bq, Wk, bk,
           Wv, bv, Wo, bo, Wvec, Wdk, bdk, Wdv, bdv):
    src = edge_index[0].astype(jnp.int32)
    dst = edge_index[1].astype(jnp.int32)

    h = jnp.arange(NH, dtype=jnp.int32)[:, None] * (3 * HD)
    d = jnp.arange(HD, dtype=jnp.int32)[None, :]
    perm = jnp.concatenate([(h + d).reshape(-1), (h + HD + d).reshape(-1),
                            (h + 2 * HD + d).reshape(-1)])
    Wv2 = Wv[perm]
    bv2 = bv[perm]
    Wdv2 = Wdv[perm]
    bdv2 = bdv[perm]

    q, kvv, vec3, vdot = _node_pre(x, vec, ln_w, ln_b, Wq, bq, Wk, bk,
                                   Wv2, bv2, Wvec)

    rd = jnp.concatenate([r_ij[:, None], d_ij], axis=1)

    CHUNKS = (79360, 79360, 79360, 81920)
    z = jnp.zeros((N, H), jnp.float32)
    agg = (z, z, z, z)
    gathered = []
    off = 0
    for ch in CHUNKS:
        gathered.append(_sc_gather(q, kvv, src[off:off + ch],
                                   dst[off:off + ch], ch))
        off += ch
    msgs = []
    off = 0
    for ch, (qg, kvvg) in zip(CHUNKS, gathered):
        msgs.append(_edge_messages(qg, kvvg, f_ij[off:off + ch],
                                   rd[off:off + ch], Wdk, bdk,
                                   Wdv2, bdv2, ch, 0))
        off += ch
    off = 0
    for ch, m in zip(CHUNKS, msgs):
        agg = _sc_scatter(m[0], m[1], m[2], m[3], *agg,
                          dst[off:off + ch], ch)
        off += ch
    a0, a1, a2, a3 = agg

    dx, dvec_flat = _node_post(a0, a1, a2, a3, vdot, vec3, Wo, bo)
    return dx, dvec_flat.reshape(N, 3, H)


# speedup vs baseline: 1.1073x; 1.0149x over previous
import functools
import jax
import jax.numpy as jnp
from jax import lax
from jax.experimental import pallas as pl
from jax.experimental.pallas import tpu as pltpu
from jax.experimental.pallas import tpu_sc as plsc

N = 10000
E = 320000
H = 128
NH = 8
HD = 16
NRBF = 32
CUT_UPPER = 5.0

BN = 1000
NB_N = N // BN
BE = 512
NB_E = E // BE


def _silu(x):
    return x * jax.nn.sigmoid(x)


def _node_pre_body(x_ref, vec_ref, lnw_ref, lnb_ref, wq_ref, bq_ref,
                   wk_ref, bk_ref, wv2_ref, bv2_ref, wvec_ref,
                   q_ref, kvv_ref, vec3_ref, vdot_ref):
    x = x_ref[...]
    mu = jnp.mean(x, axis=-1, keepdims=True)
    var = jnp.mean((x - mu) ** 2, axis=-1, keepdims=True)
    xn = (x - mu) * lax.rsqrt(var + 1e-5) * lnw_ref[...] + lnb_ref[...]
    q = jnp.dot(xn, wq_ref[...].T, preferred_element_type=jnp.float32) + bq_ref[...]
    k = jnp.dot(xn, wk_ref[...].T, preferred_element_type=jnp.float32) + bk_ref[...]
    v = jnp.dot(xn, wv2_ref[...].T, preferred_element_type=jnp.float32) + bv2_ref[...]
    vec = vec_ref[...]
    vecf = vec.reshape(BN * 3, H)
    vp = jnp.dot(vecf, wvec_ref[...].T, preferred_element_type=jnp.float32)
    vp = vp.reshape(BN, 3, 3 * H)
    vec1 = vp[:, :, :H]
    vec2 = vp[:, :, H:2 * H]
    vec3 = vp[:, :, 2 * H:]
    vdot_ref[...] = jnp.sum(vec1 * vec2, axis=1)
    vec3_ref[...] = vec3.reshape(BN, 3 * H)
    q_ref[...] = q
    kvlo = jnp.concatenate([k, v], axis=-1)
    kvhi = jnp.concatenate([vec.reshape(BN, 3 * H),
                            jnp.zeros((BN, H), jnp.float32)], axis=-1)
    lo_u = lax.bitcast_convert_type(
        kvlo.astype(jnp.bfloat16).astype(jnp.float32), jnp.uint32)
    hi_u = lax.bitcast_convert_type(
        kvhi.astype(jnp.bfloat16).astype(jnp.float32), jnp.uint32)
    kvv_ref[...] = lax.bitcast_convert_type(hi_u | (lo_u >> 16), jnp.int32)


def _node_pre(x, vec, ln_w, ln_b, Wq, bq, Wk, bk, Wv2, bv2, Wvec):
    wspec = lambda shp: pl.BlockSpec(shp, lambda i: (0,) * len(shp))
    return pl.pallas_call(
        _node_pre_body,
        grid=(NB_N,),
        in_specs=[
            pl.BlockSpec((BN, H), lambda i: (i, 0)),
            pl.BlockSpec((BN, 3, H), lambda i: (i, 0, 0)),
            wspec((H,)), wspec((H,)),
            wspec((H, H)), wspec((H,)),
            wspec((H, H)), wspec((H,)),
            wspec((3 * H, H)), wspec((3 * H,)),
            wspec((3 * H, H)),
        ],
        out_specs=[
            pl.BlockSpec((BN, H), lambda i: (i, 0)),
            pl.BlockSpec((BN, 4 * H), lambda i: (i, 0)),
            pl.BlockSpec((BN, 3 * H), lambda i: (i, 0)),
            pl.BlockSpec((BN, H), lambda i: (i, 0)),
        ],
        out_shape=[
            jax.ShapeDtypeStruct((N, H), jnp.float32),
            jax.ShapeDtypeStruct((N, 4 * H), jnp.int32),
            jax.ShapeDtypeStruct((N, 3 * H), jnp.float32),
            jax.ShapeDtypeStruct((N, H), jnp.float32),
        ],
    )(x, vec, ln_w, ln_b, Wq, bq, Wk, bk, Wv2, bv2, Wvec)


def _edge_msg_body(qg_ref, kvvg_ref, f_ref, rd_ref, wdk_ref, bdk_ref,
                   wdv2_ref, bdv2_ref, m0_ref, m1_ref, m2_ref, m3_ref):
    f = f_ref[...]
    dk = _silu(lax.dot_general(f, wdk_ref[...], (((1,), (1,)), ((), ())),
                               preferred_element_type=jnp.float32) + bdk_ref[...])
    dv = _silu(lax.dot_general(f, wdv2_ref[...], (((1,), (1,)), ((), ())),
                               preferred_element_type=jnp.float32) + bdv2_ref[...])
    qg = qg_ref[...]
    pk = lax.bitcast_convert_type(kvvg_ref[...], jnp.uint32)
    lo = lax.bitcast_convert_type(pk << 16, jnp.float32)
    hi = lax.bitcast_convert_type(pk & jnp.uint32(0xFFFF0000),
                                  jnp.float32)
    kg = lo[:, :H]
    vg = lo[:, H:4 * H]
    vecg = hi
    s = qg * kg * dk
    ri = lax.broadcasted_iota(jnp.int32, (H, H), 0) // HD
    ci = lax.broadcasted_iota(jnp.int32, (H, H), 1) // HD
    M = (ri == ci).astype(jnp.float32)
    attn = jnp.dot(s, M, preferred_element_type=jnp.float32)
    rd = rd_ref[...]
    r = rd[:, 0:1]
    cut = 0.5 * (jnp.cos(r * (jnp.pi / CUT_UPPER)) + 1.0)
    cut = cut * (r < CUT_UPPER).astype(jnp.float32)
    attn = _silu(attn) * cut
    m0_ref[...] = vg[:, :H] * dv[:, :H] * attn
    vm1 = vg[:, H:2 * H] * dv[:, H:2 * H]
    vm2 = vg[:, 2 * H:] * dv[:, 2 * H:]
    m1_ref[...] = vecg[:, :H] * vm1 + vm2 * rd[:, 1:2]
    m2_ref[...] = vecg[:, H:2 * H] * vm1 + vm2 * rd[:, 2:3]
    m3_ref[...] = vecg[:, 2 * H:3 * H] * vm1 + vm2 * rd[:, 3:4]


def _edge_messages(qg, kvvg, f_ij, rd, Wdk, bdk, Wdv2, bdv2, eh, off_b):
    wspec = lambda shp: pl.BlockSpec(shp, lambda i: (0,) * len(shp))
    espec = lambda w: pl.BlockSpec((BE, w), lambda i: (i + off_b, 0))
    return pl.pallas_call(
        _edge_msg_body,
        grid=(eh // BE,),
        in_specs=[
            espec(H), espec(4 * H), espec(NRBF), espec(4),
            wspec((H, NRBF)), wspec((H,)),
            wspec((3 * H, NRBF)), wspec((3 * H,)),
        ],
        out_specs=[pl.BlockSpec((BE, H), lambda i: (i, 0))] * 4,
        out_shape=[jax.ShapeDtypeStruct((eh, H), jnp.float32)] * 4,
    )(qg, kvvg, f_ij, rd, Wdk, bdk, Wdv2, bdv2)


NW_G = 32
CH_G = 40


def _sc_gather(q, kvv, src, dst, eh):
    epw = eh // NW_G
    npair = epw // (2 * CH_G)

    def body(q_h, kvv_h, src_h, dst_h, qg_h, kvvg_h,
             dstv, srcv, qb0, qb1, kb0, kb1,
             gsq, gsk, wsq0, wsq1, wsk0, wsk1):
        c = lax.axis_index("c")
        s = lax.axis_index("s")
        base = (s * 2 + c) * epw
        pltpu.sync_copy(dst_h.at[pl.ds(base, epw)], dstv)
        pltpu.sync_copy(src_h.at[pl.ds(base, epw)], srcv)

        def pair(g, qb, kb, wsq, wsk, b):
            off = (2 * g + b) * CH_G
            eb = base + off

            @pl.when(g > 0)
            def _():
                pltpu.make_async_copy(qb, qg_h.at[pl.ds(eb, CH_G), :], wsq).wait()
                pltpu.make_async_copy(kb, kvvg_h.at[pl.ds(eb, CH_G), :], wsk).wait()

            hq = pltpu.async_copy(q_h.at[dstv.at[pl.ds(off, CH_G)]], qb, gsq)
            hk = pltpu.async_copy(kvv_h.at[srcv.at[pl.ds(off, CH_G)]], kb, gsk)
            hq.wait()
            hk.wait()
            pltpu.async_copy(qb, qg_h.at[pl.ds(eb, CH_G), :], wsq)
            pltpu.async_copy(kb, kvvg_h.at[pl.ds(eb, CH_G), :], wsk)

        def step(g, carry):
            pair(g, qb0, kb0, wsq0, wsk0, 0)
            pair(g, qb1, kb1, wsq1, wsk1, 1)
            return carry
        lax.fori_loop(0, npair, step, 0)

        pltpu.make_async_copy(qb0, qg_h.at[pl.ds(base, CH_G), :], wsq0).wait()
        pltpu.make_async_copy(kb0, kvvg_h.at[pl.ds(base, CH_G), :], wsk0).wait()
        pltpu.make_async_copy(qb1, qg_h.at[pl.ds(base, CH_G), :], wsq1).wait()
        pltpu.make_async_copy(kb1, kvvg_h.at[pl.ds(base, CH_G), :], wsk1).wait()

    mesh = plsc.VectorSubcoreMesh(core_axis_name="c", subcore_axis_name="s")
    f = pl.kernel(
        body,
        out_type=[
            jax.ShapeDtypeStruct((eh, H), jnp.float32),
            jax.ShapeDtypeStruct((eh, 4 * H), jnp.int32),
        ],
        mesh=mesh,
        scratch_types=[
            pltpu.VMEM((epw,), jnp.int32),
            pltpu.VMEM((epw,), jnp.int32),
            pltpu.VMEM((CH_G, H), jnp.float32),
            pltpu.VMEM((CH_G, H), jnp.float32),
            pltpu.VMEM((CH_G, 4 * H), jnp.int32),
            pltpu.VMEM((CH_G, 4 * H), jnp.int32),
            pltpu.SemaphoreType.DMA,
            pltpu.SemaphoreType.DMA,
            pltpu.SemaphoreType.DMA,
            pltpu.SemaphoreType.DMA,
            pltpu.SemaphoreType.DMA,
            pltpu.SemaphoreType.DMA,
        ],
    )
    return f(q, kvv, src, dst)


NS_SC = 16
CH_S = 80
N_PAD = 10240
ZR = N_PAD // NS_SC
TAIL_R = N - (NS_SC - 1) * ZR


def _sc_scatter(m0, m1, m2, m3, i0, i1, i2, i3, dst, eh):
    ept = eh // NS_SC
    nch = ept // CH_S

    def body(m0, m1, m2, m3, i0r, i1r, i2r, i3r, dstr,
             a0, a1, a2, a3, acc, mb0, mb1, ib0, ib1, ls0, ls1):
        c = lax.axis_index("c")
        s = lax.axis_index("s")

        def do_slice(m_ref, init_ref, out_ref):
            @pl.when(s < NS_SC - 1)
            def _():
                pltpu.sync_copy(init_ref.at[pl.ds(s * ZR, ZR), :],
                                acc.at[pl.ds(s * ZR, ZR), :])

            @pl.when(s == NS_SC - 1)
            def _():
                pltpu.sync_copy(init_ref.at[pl.ds(s * ZR, TAIL_R), :],
                                acc.at[pl.ds(s * ZR, TAIL_R), :])
            plsc.subcore_barrier()

            tbase = s * ept

            def load(i, mb, ib, ls):
                eb = tbase + i * CH_S
                pltpu.async_copy(dstr.at[pl.ds(eb, CH_S)], ib, ls)
                pltpu.async_copy(m_ref.at[pl.ds(eb, CH_S), :], mb, ls)

            def drain(i, mb, ib, ls):
                eb = tbase + i * CH_S
                pltpu.make_async_copy(dstr.at[pl.ds(eb, CH_S)], ib, ls).wait()
                pltpu.make_async_copy(m_ref.at[pl.ds(eb, CH_S), :], mb, ls).wait()

            load(0, mb0, ib0, ls0)
            load(1, mb1, ib1, ls1)

            def step(i, mb, ib, ls, b):
                drain(2 * i + b, mb, ib, ls)
                pltpu.sync_copy(mb, acc.at[ib], add=True)

                @pl.when(i < nch // 2 - 1)
                def _():
                    load(2 * i + b + 2, mb, ib, ls)

            def _chunk(i, carry):
                step(i, mb0, ib0, ls0, 0)
                step(i, mb1, ib1, ls1, 1)
                return carry
            lax.fori_loop(0, nch // 2, _chunk, 0)
            plsc.subcore_barrier()

            @pl.when(s < NS_SC - 1)
            def _full():
                pltpu.sync_copy(acc.at[pl.ds(s * ZR, ZR), :],
                                out_ref.at[pl.ds(s * ZR, ZR), :])

            @pl.when(s == NS_SC - 1)
            def _tail():
                pltpu.sync_copy(acc.at[pl.ds(s * ZR, TAIL_R), :],
                                out_ref.at[pl.ds(s * ZR, TAIL_R), :])

        @pl.when(c == 0)
        def _():
            do_slice(m0, i0r, a0)
            do_slice(m2, i2r, a2)

        @pl.when(c == 1)
        def _():
            do_slice(m1, i1r, a1)
            do_slice(m3, i3r, a3)

    mesh = plsc.VectorSubcoreMesh(core_axis_name="c", subcore_axis_name="s")
    f = pl.kernel(
        body,
        out_type=[jax.ShapeDtypeStruct((N, H), jnp.float32)] * 4,
        mesh=mesh,
        scratch_types=[
            pltpu.VMEM_SHARED((N_PAD, H), jnp.float32),
            pltpu.VMEM((CH_S, H), jnp.float32),
            pltpu.VMEM((CH_S, H), jnp.float32),
            pltpu.VMEM((CH_S,), jnp.int32),
            pltpu.VMEM((CH_S,), jnp.int32),
            pltpu.SemaphoreType.DMA,
            pltpu.SemaphoreType.DMA,
        ],
    )
    return f(m0, m1, m2, m3, i0, i1, i2, i3, dst)


def _node_post_body(a0_ref, a1_ref, a2_ref, a3_ref, vdot_ref, vec3_ref,
                    wo_ref, bo_ref, dx_ref, dvec_ref):
    o = jnp.dot(a0_ref[...], wo_ref[...].T, preferred_element_type=jnp.float32) + bo_ref[...]
    o1 = o[:, :H]
    o2 = o[:, H:2 * H]
    o3 = o[:, 2 * H:]
    dx_ref[...] = vdot_ref[...] * o2 + o3
    vec3 = vec3_ref[...]
    d1 = vec3[:, :H] * o1 + a1_ref[...]
    d2 = vec3[:, H:2 * H] * o1 + a2_ref[...]
    d3 = vec3[:, 2 * H:] * o1 + a3_ref[...]
    dvec_ref[...] = jnp.concatenate([d1, d2, d3], axis=-1)


def _node_post(a0, a1, a2, a3, vdot, vec3, Wo, bo):
    wspec = lambda shp: pl.BlockSpec(shp, lambda i: (0,) * len(shp))
    nspec = lambda w: pl.BlockSpec((BN, w), lambda i: (i, 0))
    return pl.pallas_call(
        _node_post_body,
        grid=(NB_N,),
        in_specs=[
            nspec(H), nspec(H), nspec(H), nspec(H), nspec(H), nspec(3 * H),
            wspec((3 * H, H)), wspec((3 * H,)),
        ],
        out_specs=[nspec(H), nspec(3 * H)],
        out_shape=[
            jax.ShapeDtypeStruct((N, H), jnp.float32),
            jax.ShapeDtypeStruct((N, 3 * H), jnp.float32),
        ],
    )(a0, a1, a2, a3, vdot, vec3, Wo, bo)


def kernel(x, vec, edge_index, r_ij, f_ij, d_ij, ln_w, ln_b, Wq, bq, Wk, bk,
           Wv, bv, Wo, bo, Wvec, Wdk, bdk, Wdv, bdv):
    src = edge_index[0].astype(jnp.int32)
    dst = edge_index[1].astype(jnp.int32)

    h = jnp.arange(NH, dtype=jnp.int32)[:, None] * (3 * HD)
    d = jnp.arange(HD, dtype=jnp.int32)[None, :]
    perm = jnp.concatenate([(h + d).reshape(-1), (h + HD + d).reshape(-1),
                            (h + 2 * HD + d).reshape(-1)])
    Wv2 = Wv[perm]
    bv2 = bv[perm]
    Wdv2 = Wdv[perm]
    bdv2 = bdv[perm]

    q, kvv, vec3, vdot = _node_pre(x, vec, ln_w, ln_b, Wq, bq, Wk, bk,
                                   Wv2, bv2, Wvec)

    rd = jnp.concatenate([r_ij[:, None], d_ij], axis=1)

    CHUNKS = (40960, 40960, 40960, 40960, 40960, 38400, 38400, 38400)
    z = jnp.zeros((N, H), jnp.float32)
    agg = (z, z, z, z)
    gathered = []
    off = 0
    for ch in CHUNKS:
        gathered.append(_sc_gather(q, kvv, src[off:off + ch],
                                   dst[off:off + ch], ch))
        off += ch
    msgs = []
    off = 0
    for ch, (qg, kvvg) in zip(CHUNKS, gathered):
        msgs.append(_edge_messages(qg, kvvg, f_ij[off:off + ch],
                                   rd[off:off + ch], Wdk, bdk,
                                   Wdv2, bdv2, ch, 0))
        off += ch
    off = 0
    for ch, m in zip(CHUNKS, msgs):
        agg = _sc_scatter(m[0], m[1], m[2], m[3], *agg,
                          dst[off:off + ch], ch)
        off += ch
    a0, a1, a2, a3 = agg

    dx, dvec_flat = _node_post(a0, a1, a2, a3, vdot, vec3, Wo, bo)
    return dx, dvec_flat.reshape(N, 3, H)


# speedup vs baseline: 1.1082x; 1.0008x over previous
import jax
import jax.numpy as jnp
from jax import lax
from jax.experimental import pallas as pl
from jax.experimental.pallas import tpu as pltpu
from jax.experimental.pallas import tpu_sc as plsc

N = 10000
E = 320000
H = 128
NH = 8
HD = 16
NRBF = 32
CUT_UPPER = 5.0

BN = 1000
NB_N = N // BN
BE = 512
NB_E = E // BE


def _silu(x):
    return x * jax.nn.sigmoid(x)


def _node_pre_body(x_ref, vec_ref, lnw_ref, lnb_ref, wq_ref, bq_ref,
                   wk_ref, bk_ref, wv2_ref, bv2_ref, wvec_ref,
                   q_ref, kvv_ref, vec3_ref, vdot_ref):
    x = x_ref[...]
    mu = jnp.mean(x, axis=-1, keepdims=True)
    var = jnp.mean((x - mu) ** 2, axis=-1, keepdims=True)
    xn = (x - mu) * lax.rsqrt(var + 1e-5) * lnw_ref[...] + lnb_ref[...]
    q = jnp.dot(xn, wq_ref[...].T, preferred_element_type=jnp.float32) + bq_ref[...]
    k = jnp.dot(xn, wk_ref[...].T, preferred_element_type=jnp.float32) + bk_ref[...]
    v = jnp.dot(xn, wv2_ref[...].T, preferred_element_type=jnp.float32) + bv2_ref[...]
    vec = vec_ref[...]
    vecf = vec.reshape(BN * 3, H)
    vp = jnp.dot(vecf, wvec_ref[...].T, preferred_element_type=jnp.float32)
    vp = vp.reshape(BN, 3, 3 * H)
    vec1 = vp[:, :, :H]
    vec2 = vp[:, :, H:2 * H]
    vec3 = vp[:, :, 2 * H:]
    vdot_ref[...] = jnp.sum(vec1 * vec2, axis=1)
    vec3_ref[...] = vec3.reshape(BN, 3 * H)
    q_ref[...] = q
    kvlo = jnp.concatenate([k, v], axis=-1)
    kvhi = jnp.concatenate([vec.reshape(BN, 3 * H),
                            jnp.zeros((BN, H), jnp.float32)], axis=-1)
    lo_u = lax.bitcast_convert_type(
        kvlo.astype(jnp.bfloat16).astype(jnp.float32), jnp.uint32)
    hi_u = lax.bitcast_convert_type(
        kvhi.astype(jnp.bfloat16).astype(jnp.float32), jnp.uint32)
    kvv_ref[...] = lax.bitcast_convert_type(hi_u | (lo_u >> 16), jnp.int32)


def _node_pre(x, vec, ln_w, ln_b, Wq, bq, Wk, bk, Wv2, bv2, Wvec):
    wspec = lambda shp: pl.BlockSpec(shp, lambda i: (0,) * len(shp))
    return pl.pallas_call(
        _node_pre_body,
        grid=(NB_N,),
        in_specs=[
            pl.BlockSpec((BN, H), lambda i: (i, 0)),
            pl.BlockSpec((BN, 3, H), lambda i: (i, 0, 0)),
            wspec((H,)), wspec((H,)),
            wspec((H, H)), wspec((H,)),
            wspec((H, H)), wspec((H,)),
            wspec((3 * H, H)), wspec((3 * H,)),
            wspec((3 * H, H)),
        ],
        out_specs=[
            pl.BlockSpec((BN, H), lambda i: (i, 0)),
            pl.BlockSpec((BN, 4 * H), lambda i: (i, 0)),
            pl.BlockSpec((BN, 3 * H), lambda i: (i, 0)),
            pl.BlockSpec((BN, H), lambda i: (i, 0)),
        ],
        out_shape=[
            jax.ShapeDtypeStruct((N, H), jnp.float32),
            jax.ShapeDtypeStruct((N, 4 * H), jnp.int32),
            jax.ShapeDtypeStruct((N, 3 * H), jnp.float32),
            jax.ShapeDtypeStruct((N, H), jnp.float32),
        ],
    )(x, vec, ln_w, ln_b, Wq, bq, Wk, bk, Wv2, bv2, Wvec)


def _edge_msg_body(qg_ref, kvvg_ref, f_ref, rd_ref, wdk_ref, bdk_ref,
                   wdv2_ref, bdv2_ref, m0_ref, m1_ref, m2_ref, m3_ref):
    f = f_ref[...]
    dk = _silu(lax.dot_general(f, wdk_ref[...], (((1,), (1,)), ((), ())),
                               preferred_element_type=jnp.float32) + bdk_ref[...])
    dv = _silu(lax.dot_general(f, wdv2_ref[...], (((1,), (1,)), ((), ())),
                               preferred_element_type=jnp.float32) + bdv2_ref[...])
    qg = qg_ref[...]
    pk = lax.bitcast_convert_type(kvvg_ref[...], jnp.uint32)
    lo = lax.bitcast_convert_type(pk << 16, jnp.float32)
    hi = lax.bitcast_convert_type(pk & jnp.uint32(0xFFFF0000),
                                  jnp.float32)
    kg = lo[:, :H]
    vg = lo[:, H:4 * H]
    vecg = hi
    s = qg * kg * dk
    ri = lax.broadcasted_iota(jnp.int32, (H, H), 0) // HD
    ci = lax.broadcasted_iota(jnp.int32, (H, H), 1) // HD
    M = (ri == ci).astype(jnp.float32)
    attn = jnp.dot(s, M, preferred_element_type=jnp.float32)
    rd = rd_ref[...]
    r = rd[:, 0:1]
    cut = 0.5 * (jnp.cos(r * (jnp.pi / CUT_UPPER)) + 1.0)
    cut = cut * (r < CUT_UPPER).astype(jnp.float32)
    attn = _silu(attn) * cut
    m0_ref[...] = vg[:, :H] * dv[:, :H] * attn
    vm1 = vg[:, H:2 * H] * dv[:, H:2 * H]
    vm2 = vg[:, 2 * H:] * dv[:, 2 * H:]
    m1_ref[...] = vecg[:, :H] * vm1 + vm2 * rd[:, 1:2]
    m2_ref[...] = vecg[:, H:2 * H] * vm1 + vm2 * rd[:, 2:3]
    m3_ref[...] = vecg[:, 2 * H:3 * H] * vm1 + vm2 * rd[:, 3:4]


def _edge_messages(qg, kvvg, f_ij, rd, Wdk, bdk, Wdv2, bdv2, eh, off_b):
    wspec = lambda shp: pl.BlockSpec(shp, lambda i: (0,) * len(shp))
    espec = lambda w: pl.BlockSpec((BE, w), lambda i: (i + off_b, 0))
    return pl.pallas_call(
        _edge_msg_body,
        grid=(eh // BE,),
        in_specs=[
            espec(H), espec(4 * H), espec(NRBF), espec(4),
            wspec((H, NRBF)), wspec((H,)),
            wspec((3 * H, NRBF)), wspec((3 * H,)),
        ],
        out_specs=[pl.BlockSpec((BE, H), lambda i: (i, 0))] * 4,
        out_shape=[jax.ShapeDtypeStruct((eh, H), jnp.float32)] * 4,
    )(qg, kvvg, f_ij, rd, Wdk, bdk, Wdv2, bdv2)


NW_G = 32
CH_G = 40


def _sc_gather(q, kvv, src, dst, eh):
    epw = eh // NW_G
    npair = epw // (2 * CH_G)

    def body(q_h, kvv_h, src_h, dst_h, qg_h, kvvg_h,
             dstv, srcv, qb0, qb1, kb0, kb1,
             gsq, gsk, wsq0, wsq1, wsk0, wsk1):
        c = lax.axis_index("c")
        s = lax.axis_index("s")
        base = (s * 2 + c) * epw
        pltpu.sync_copy(dst_h.at[pl.ds(base, epw)], dstv)
        pltpu.sync_copy(src_h.at[pl.ds(base, epw)], srcv)

        def pair(g, qb, kb, wsq, wsk, b):
            off = (2 * g + b) * CH_G
            eb = base + off

            @pl.when(g > 0)
            def _():
                pltpu.make_async_copy(qb, qg_h.at[pl.ds(eb, CH_G), :], wsq).wait()
                pltpu.make_async_copy(kb, kvvg_h.at[pl.ds(eb, CH_G), :], wsk).wait()

            hq = pltpu.async_copy(q_h.at[dstv.at[pl.ds(off, CH_G)]], qb, gsq)
            hk = pltpu.async_copy(kvv_h.at[srcv.at[pl.ds(off, CH_G)]], kb, gsk)
            hq.wait()
            hk.wait()
            pltpu.async_copy(qb, qg_h.at[pl.ds(eb, CH_G), :], wsq)
            pltpu.async_copy(kb, kvvg_h.at[pl.ds(eb, CH_G), :], wsk)

        def step(g, carry):
            pair(g, qb0, kb0, wsq0, wsk0, 0)
            pair(g, qb1, kb1, wsq1, wsk1, 1)
            return carry
        lax.fori_loop(0, npair, step, 0)

        pltpu.make_async_copy(qb0, qg_h.at[pl.ds(base, CH_G), :], wsq0).wait()
        pltpu.make_async_copy(kb0, kvvg_h.at[pl.ds(base, CH_G), :], wsk0).wait()
        pltpu.make_async_copy(qb1, qg_h.at[pl.ds(base, CH_G), :], wsq1).wait()
        pltpu.make_async_copy(kb1, kvvg_h.at[pl.ds(base, CH_G), :], wsk1).wait()

    mesh = plsc.VectorSubcoreMesh(core_axis_name="c", subcore_axis_name="s")
    f = pl.kernel(
        body,
        out_type=[
            jax.ShapeDtypeStruct((eh, H), jnp.float32),
            jax.ShapeDtypeStruct((eh, 4 * H), jnp.int32),
        ],
        mesh=mesh,
        scratch_types=[
            pltpu.VMEM((epw,), jnp.int32),
            pltpu.VMEM((epw,), jnp.int32),
            pltpu.VMEM((CH_G, H), jnp.float32),
            pltpu.VMEM((CH_G, H), jnp.float32),
            pltpu.VMEM((CH_G, 4 * H), jnp.int32),
            pltpu.VMEM((CH_G, 4 * H), jnp.int32),
            pltpu.SemaphoreType.DMA,
            pltpu.SemaphoreType.DMA,
            pltpu.SemaphoreType.DMA,
            pltpu.SemaphoreType.DMA,
            pltpu.SemaphoreType.DMA,
            pltpu.SemaphoreType.DMA,
        ],
    )
    return f(q, kvv, src, dst)


NS_SC = 16
CH_S = 80
N_PAD = 10240
ZR = N_PAD // NS_SC
TAIL_R = N - (NS_SC - 1) * ZR


def _sc_scatter(m0, m1, m2, m3, i0, i1, i2, i3, dst, eh):
    ept = eh // NS_SC
    nch = ept // CH_S

    def body(m0, m1, m2, m3, i0r, i1r, i2r, i3r, dstr,
             a0, a1, a2, a3, acc, mb0, mb1, ib0, ib1, ls0, ls1):
        c = lax.axis_index("c")
        s = lax.axis_index("s")

        def do_slice(m_ref, init_ref, out_ref):
            @pl.when(s < NS_SC - 1)
            def _():
                pltpu.sync_copy(init_ref.at[pl.ds(s * ZR, ZR), :],
                                acc.at[pl.ds(s * ZR, ZR), :])

            @pl.when(s == NS_SC - 1)
            def _():
                pltpu.sync_copy(init_ref.at[pl.ds(s * ZR, TAIL_R), :],
                                acc.at[pl.ds(s * ZR, TAIL_R), :])
            plsc.subcore_barrier()

            tbase = s * ept

            def load(i, mb, ib, ls):
                eb = tbase + i * CH_S
                pltpu.async_copy(dstr.at[pl.ds(eb, CH_S)], ib, ls)
                pltpu.async_copy(m_ref.at[pl.ds(eb, CH_S), :], mb, ls)

            def drain(i, mb, ib, ls):
                eb = tbase + i * CH_S
                pltpu.make_async_copy(dstr.at[pl.ds(eb, CH_S)], ib, ls).wait()
                pltpu.make_async_copy(m_ref.at[pl.ds(eb, CH_S), :], mb, ls).wait()

            load(0, mb0, ib0, ls0)
            load(1, mb1, ib1, ls1)

            def step(i, mb, ib, ls, b):
                drain(2 * i + b, mb, ib, ls)
                pltpu.sync_copy(mb, acc.at[ib], add=True)

                @pl.when(i < nch // 2 - 1)
                def _():
                    load(2 * i + b + 2, mb, ib, ls)

            def _chunk(i, carry):
                step(i, mb0, ib0, ls0, 0)
                step(i, mb1, ib1, ls1, 1)
                return carry
            lax.fori_loop(0, nch // 2, _chunk, 0)
            plsc.subcore_barrier()

            @pl.when(s < NS_SC - 1)
            def _full():
                pltpu.sync_copy(acc.at[pl.ds(s * ZR, ZR), :],
                                out_ref.at[pl.ds(s * ZR, ZR), :])

            @pl.when(s == NS_SC - 1)
            def _tail():
                pltpu.sync_copy(acc.at[pl.ds(s * ZR, TAIL_R), :],
                                out_ref.at[pl.ds(s * ZR, TAIL_R), :])

        @pl.when(c == 0)
        def _():
            do_slice(m0, i0r, a0)
            do_slice(m2, i2r, a2)

        @pl.when(c == 1)
        def _():
            do_slice(m1, i1r, a1)
            do_slice(m3, i3r, a3)

    mesh = plsc.VectorSubcoreMesh(core_axis_name="c", subcore_axis_name="s")
    f = pl.kernel(
        body,
        out_type=[jax.ShapeDtypeStruct((N, H), jnp.float32)] * 4,
        mesh=mesh,
        scratch_types=[
            pltpu.VMEM_SHARED((N_PAD, H), jnp.float32),
            pltpu.VMEM((CH_S, H), jnp.float32),
            pltpu.VMEM((CH_S, H), jnp.float32),
            pltpu.VMEM((CH_S,), jnp.int32),
            pltpu.VMEM((CH_S,), jnp.int32),
            pltpu.SemaphoreType.DMA,
            pltpu.SemaphoreType.DMA,
        ],
    )
    return f(m0, m1, m2, m3, i0, i1, i2, i3, dst)


def _node_post_body(a0_ref, a1_ref, a2_ref, a3_ref, vdot_ref, vec3_ref,
                    wo_ref, bo_ref, dx_ref, dvec_ref):
    o = jnp.dot(a0_ref[...], wo_ref[...].T, preferred_element_type=jnp.float32) + bo_ref[...]
    o1 = o[:, :H]
    o2 = o[:, H:2 * H]
    o3 = o[:, 2 * H:]
    dx_ref[...] = vdot_ref[...] * o2 + o3
    vec3 = vec3_ref[...]
    d1 = vec3[:, :H] * o1 + a1_ref[...]
    d2 = vec3[:, H:2 * H] * o1 + a2_ref[...]
    d3 = vec3[:, 2 * H:] * o1 + a3_ref[...]
    dvec_ref[...] = jnp.concatenate([d1, d2, d3], axis=-1)


def _node_post(a0, a1, a2, a3, vdot, vec3, Wo, bo):
    wspec = lambda shp: pl.BlockSpec(shp, lambda i: (0,) * len(shp))
    nspec = lambda w: pl.BlockSpec((BN, w), lambda i: (i, 0))
    return pl.pallas_call(
        _node_post_body,
        grid=(NB_N,),
        in_specs=[
            nspec(H), nspec(H), nspec(H), nspec(H), nspec(H), nspec(3 * H),
            wspec((3 * H, H)), wspec((3 * H,)),
        ],
        out_specs=[nspec(H), nspec(3 * H)],
        out_shape=[
            jax.ShapeDtypeStruct((N, H), jnp.float32),
            jax.ShapeDtypeStruct((N, 3 * H), jnp.float32),
        ],
    )(a0, a1, a2, a3, vdot, vec3, Wo, bo)


def kernel(x, vec, edge_index, r_ij, f_ij, d_ij, ln_w, ln_b, Wq, bq, Wk, bk,
           Wv, bv, Wo, bo, Wvec, Wdk, bdk, Wdv, bdv):
    src = edge_index[0].astype(jnp.int32)
    dst = edge_index[1].astype(jnp.int32)

    h = jnp.arange(NH, dtype=jnp.int32)[:, None] * (3 * HD)
    d = jnp.arange(HD, dtype=jnp.int32)[None, :]
    perm = jnp.concatenate([(h + d).reshape(-1), (h + HD + d).reshape(-1),
                            (h + 2 * HD + d).reshape(-1)])
    Wv2 = Wv[perm]
    bv2 = bv[perm]
    Wdv2 = Wdv[perm]
    bdv2 = bdv[perm]

    q, kvv, vec3, vdot = _node_pre(x, vec, ln_w, ln_b, Wq, bq, Wk, bk,
                                   Wv2, bv2, Wvec)

    rd = jnp.concatenate([r_ij[:, None], d_ij], axis=1)

    CHUNKS = (40960, 40960, 40960, 40960, 40960, 38400, 38400, 38400)
    z = jnp.zeros((N, H), jnp.float32)
    agg = (z, z, z, z)
    gathered = []
    off = 0
    for ch in CHUNKS:
        gathered.append(_sc_gather(q, kvv, src[off:off + ch],
                                   dst[off:off + ch], ch))
        off += ch
    msgs = []
    off = 0
    for ch, (qg, kvvg) in zip(CHUNKS, gathered):
        msgs.append(_edge_messages(qg, kvvg, f_ij[off:off + ch],
                                   rd[off:off + ch], Wdk, bdk,
                                   Wdv2, bdv2, ch, 0))
        off += ch
    off = 0
    for ch, m in zip(CHUNKS, msgs):
        agg = _sc_scatter(m[0], m[1], m[2], m[3], *agg,
                          dst[off:off + ch], ch)
        off += ch
    a0, a1, a2, a3 = agg

    dx, dvec_flat = _node_post(a0, a1, a2, a3, vdot, vec3, Wo, bo)
    return dx, dvec_flat.reshape(N, 3, H)
